# baseline (speedup 1.0000x reference)
import functools

import jax
import jax.numpy as jnp
from jax import lax
from jax.experimental import pallas as pl
from jax.experimental.pallas import tpu as pltpu
from jax.experimental.pallas import tpu_sc as plsc

_N = 100000
_E = 200000
_H = 128

_NC = 2
_NS = 16
_NW = _NC * _NS
_R = (2 * _E) // 128
_RB = 96
_RL = _R - _NW * _RB
_RXF = _RL // 8
_RXT = _RL - 8 * _RXF
_NPAD = 100352
_ZS = _NPAD // _NS

_BN = 7168
_G = _NPAD // _BN


@functools.cache
def _sc_bincount():
    @functools.partial(
        pl.kernel,
        out_type=jax.ShapeDtypeStruct((_NC * _NPAD,), jnp.int32),
        mesh=plsc.VectorSubcoreMesh(core_axis_name="c", subcore_axis_name="s"),
        scratch_types=[
            pltpu.VMEM((_RB + 8, 128), jnp.int32),
            pltpu.VMEM((128,), jnp.int32),
            pltpu.VMEM((_ZS,), jnp.int32),
            pltpu.VMEM_SHARED((_NPAD,), jnp.int32),
            pltpu.SemaphoreType.DMA,
        ],
    )
    def sc_bincount(edges_hbm, out_hbm, idx_v, ones_v, zeros_v, counts_sh, sem):
        c = lax.axis_index("c")
        s = lax.axis_index("s")
        w = c * _NS + s

        pltpu.async_copy(edges_hbm.at[pl.ds(w * _RB, _RB)], idx_v.at[pl.ds(0, _RB)], sem)

        @pl.when(w < _RXF)
        def _():
            pltpu.async_copy(edges_hbm.at[pl.ds(_NW * _RB + 8 * w, 8)],
                             idx_v.at[pl.ds(_RB, 8)], sem)

        @pl.when(w == _RXF)
        def _():
            pltpu.async_copy(edges_hbm.at[pl.ds(_NW * _RB + 8 * _RXF, _RXT)],
                             idx_v.at[pl.ds(_RB, _RXT)], sem)

        o16 = jnp.ones((16,), jnp.int32)
        for i in range(8):
            ones_v[pl.ds(i * 16, 16)] = o16

        z16 = jnp.zeros((16,), jnp.int32)

        @plsc.parallel_loop(0, _ZS // 16)
        def _(i):
            zeros_v[pl.ds(i * 16, 16)] = z16

        nr = _RB + jnp.where(w < _RXF, 8, jnp.where(w == _RXF, _RXT, 0))

        pltpu.sync_copy(zeros_v, counts_sh.at[pl.ds(s * _ZS, _ZS)])
        pltpu.make_async_copy(edges_hbm.at[pl.ds(w * _RB, _RB)],
                              idx_v.at[pl.ds(0, _RB)], sem).wait()

        @pl.when(w < _RXF)
        def _():
            pltpu.make_async_copy(edges_hbm.at[pl.ds(_NW * _RB + 8 * w, 8)],
                                  idx_v.at[pl.ds(_RB, 8)], sem).wait()

        @pl.when(w == _RXF)
        def _():
            pltpu.make_async_copy(edges_hbm.at[pl.ds(_NW * _RB + 8 * _RXF, _RXT)],
                                  idx_v.at[pl.ds(_RB, _RXT)], sem).wait()

        plsc.subcore_barrier()

        @plsc.parallel_loop(0, nr, unroll=4)
        def _(j):
            pltpu.async_copy(ones_v, counts_sh.at[idx_v.at[j]], sem, add=True)

        def drain(j, carry):
            pltpu.make_async_copy(ones_v, counts_sh.at[idx_v.at[j]], sem).wait()
            return carry

        lax.fori_loop(0, nr, drain, 0)

        plsc.subcore_barrier()
        pltpu.sync_copy(counts_sh.at[pl.ds(s * _ZS, _ZS)],
                        out_hbm.at[pl.ds(c * _NPAD + s * _ZS, _ZS)])

    return sc_bincount


def _tc_body(xt_ref, w_ref, b_ref, t_ref, c0_ref, c1_ref, o_ref):
    deg = (c0_ref[...] + c1_ref[...]).reshape(1, _BN)
    d = jnp.minimum(deg >> 1, 9)
    iot = lax.broadcasted_iota(jnp.int32, (10, _BN), 0)
    oh_t = (iot == d).astype(jnp.float32)
    xd = lax.dot_general(oh_t, t_ref[...], (((0,), (0,)), ((), ())),
                         preferred_element_type=jnp.float32)
    h = lax.dot_general(xt_ref[...], w_ref[...], (((0,), (0,)), ((), ())),
                        preferred_element_type=jnp.float32)
    o_ref[...] = h + xd + b_ref[...]


def kernel(x, edge_idx, W_feat, b_feat, degree_table):
    edges = edge_idx.reshape(_R, 128)

    counts = _sc_bincount()(edges)
    xt = x.T
    b2 = b_feat.reshape(1, _H)

    out = pl.pallas_call(
        _tc_body,
        grid=(_G,),
        in_specs=[
            pl.BlockSpec((7, _BN), lambda i: (0, i)),
            pl.BlockSpec((7, _H), lambda i: (0, 0)),
            pl.BlockSpec((1, _H), lambda i: (0, 0)),
            pl.BlockSpec((10, _H), lambda i: (0, 0)),
            pl.BlockSpec((_BN,), lambda i: (i,)),
            pl.BlockSpec((_BN,), lambda i: (i + _G,)),
        ],
        out_specs=pl.BlockSpec((_BN, _H), lambda i: (i, 0)),
        out_shape=jax.ShapeDtypeStruct((_N, _H), jnp.float32),
    )(xt, W_feat, b2, degree_table, counts, counts)
    return out


# speedup vs baseline: 1.0236x; 1.0236x over previous
import functools

import jax
import jax.numpy as jnp
from jax import lax
from jax.experimental import pallas as pl
from jax.experimental.pallas import tpu as pltpu
from jax.experimental.pallas import tpu_sc as plsc

_N = 100000
_E = 200000
_H = 128

_NC = 2
_NS = 16
_NW = _NC * _NS
_R = (2 * _E) // 128
_RB = 96
_RL = _R - _NW * _RB
_RXF = _RL // 8
_RXT = _RL - 8 * _RXF
_NPAD = 100352
_ZS = _NPAD // _NS

_BN = 14336
_G = _NPAD // _BN


@functools.cache
def _sc_bincount():
    @functools.partial(
        pl.kernel,
        out_type=jax.ShapeDtypeStruct((_NC * _NPAD,), jnp.int32),
        mesh=plsc.VectorSubcoreMesh(core_axis_name="c", subcore_axis_name="s"),
        scratch_types=[
            pltpu.VMEM((_RB + 8, 128), jnp.int32),
            pltpu.VMEM((128,), jnp.int32),
            pltpu.VMEM((_ZS,), jnp.int32),
            pltpu.VMEM_SHARED((_NPAD,), jnp.int32),
            pltpu.SemaphoreType.DMA,
        ],
    )
    def sc_bincount(edges_hbm, out_hbm, idx_v, ones_v, zeros_v, counts_sh, sem):
        c = lax.axis_index("c")
        s = lax.axis_index("s")
        w = c * _NS + s

        pltpu.async_copy(edges_hbm.at[pl.ds(w * _RB, _RB)], idx_v.at[pl.ds(0, _RB)], sem)

        @pl.when(w < _RXF)
        def _():
            pltpu.async_copy(edges_hbm.at[pl.ds(_NW * _RB + 8 * w, 8)],
                             idx_v.at[pl.ds(_RB, 8)], sem)

        @pl.when(w == _RXF)
        def _():
            pltpu.async_copy(edges_hbm.at[pl.ds(_NW * _RB + 8 * _RXF, _RXT)],
                             idx_v.at[pl.ds(_RB, _RXT)], sem)

        o16 = jnp.ones((16,), jnp.int32)
        for i in range(8):
            ones_v[pl.ds(i * 16, 16)] = o16

        z16 = jnp.zeros((16,), jnp.int32)

        @plsc.parallel_loop(0, _ZS // 16)
        def _(i):
            zeros_v[pl.ds(i * 16, 16)] = z16

        nr = _RB + jnp.where(w < _RXF, 8, jnp.where(w == _RXF, _RXT, 0))

        pltpu.sync_copy(zeros_v, counts_sh.at[pl.ds(s * _ZS, _ZS)])
        pltpu.make_async_copy(edges_hbm.at[pl.ds(w * _RB, _RB)],
                              idx_v.at[pl.ds(0, _RB)], sem).wait()

        @pl.when(w < _RXF)
        def _():
            pltpu.make_async_copy(edges_hbm.at[pl.ds(_NW * _RB + 8 * w, 8)],
                                  idx_v.at[pl.ds(_RB, 8)], sem).wait()

        @pl.when(w == _RXF)
        def _():
            pltpu.make_async_copy(edges_hbm.at[pl.ds(_NW * _RB + 8 * _RXF, _RXT)],
                                  idx_v.at[pl.ds(_RB, _RXT)], sem).wait()

        plsc.subcore_barrier()

        @plsc.parallel_loop(0, nr, unroll=8)
        def _(j):
            pltpu.async_copy(ones_v, counts_sh.at[idx_v.at[j]], sem, add=True)

        def drain(j, carry):
            pltpu.make_async_copy(ones_v, counts_sh.at[idx_v.at[j]], sem).wait()
            return carry

        lax.fori_loop(0, nr, drain, 0)

        plsc.subcore_barrier()
        pltpu.sync_copy(counts_sh.at[pl.ds(s * _ZS, _ZS)],
                        out_hbm.at[pl.ds(c * _NPAD + s * _ZS, _ZS)])

    return sc_bincount


def _tc_body(xt_ref, w_ref, b_ref, t_ref, c0_ref, c1_ref, o_ref):
    deg = (c0_ref[...] + c1_ref[...]).reshape(1, _BN)
    d = jnp.minimum(deg >> 1, 9)
    iot = lax.broadcasted_iota(jnp.int32, (10, _BN), 0)
    oh_t = (iot == d).astype(jnp.float32)
    xd = lax.dot_general(oh_t, t_ref[...], (((0,), (0,)), ((), ())),
                         preferred_element_type=jnp.float32)
    h = lax.dot_general(xt_ref[...], w_ref[...], (((0,), (0,)), ((), ())),
                        preferred_element_type=jnp.float32)
    o_ref[...] = h + xd + b_ref[...]


def kernel(x, edge_idx, W_feat, b_feat, degree_table):
    edges = edge_idx.reshape(_R, 128)

    counts = _sc_bincount()(edges)
    xt = x.T
    b2 = b_feat.reshape(1, _H)

    out = pl.pallas_call(
        _tc_body,
        grid=(_G,),
        in_specs=[
            pl.BlockSpec((7, _BN), lambda i: (0, i)),
            pl.BlockSpec((7, _H), lambda i: (0, 0)),
            pl.BlockSpec((1, _H), lambda i: (0, 0)),
            pl.BlockSpec((10, _H), lambda i: (0, 0)),
            pl.BlockSpec((_BN,), lambda i: (i,)),
            pl.BlockSpec((_BN,), lambda i: (i + _G,)),
        ],
        out_specs=pl.BlockSpec((_BN, _H), lambda i: (i, 0)),
        out_shape=jax.ShapeDtypeStruct((_N, _H), jnp.float32),
    )(xt, W_feat, b2, degree_table, counts, counts)
    return out
